# Initial kernel scaffold; baseline (speedup 1.0000x reference)
import jax
import jax.numpy as jnp
from jax.experimental import pallas as pl


def kernel(x, edge_index, edge_weight, W1, b1, W2, b2, Wh1, bh1, Wh2, bh2):
    raise NotImplementedError("write your pallas kernel here")


# speedup vs baseline: 4.4121x; 4.4121x over previous
import jax
import jax.numpy as jnp
from jax import lax
from jax.experimental import pallas as pl
from jax.experimental.pallas import tpu as pltpu
from jax.experimental.pallas import tpu_sc as plsc

N = 99999
IN = 128
H = 48
E = 1599984

NC = 2
NS = 16
NW = NC * NS

BLK = 128
CPB = 8
CHUNK = CPB * BLK
CHUNKS_PER_TILE = 49
EDGES_PER_TILE = CHUNK * CHUNKS_PER_TILE
E_PAD = EDGES_PER_TILE * NW
EROWS = E_PAD // BLK

NP = 104448
SLICE = NP // NS

F = 4
P = H // F

_mesh = plsc.VectorSubcoreMesh(core_axis_name="c", subcore_axis_name="s")


def _deg_body(dstH, wH, z, deg_out, dtab, d8, w8):
    c = lax.axis_index("c")
    s = lax.axis_index("s")
    wid = s * NC + c
    pltpu.sync_copy(z, dtab.at[pl.ds(s * SLICE, SLICE)])
    plsc.subcore_barrier()

    def chunk(k, carry):
        row = (wid * CHUNKS_PER_TILE + k) * CPB
        pltpu.sync_copy(dstH.at[pl.ds(row, CPB)], d8)
        pltpu.sync_copy(wH.at[pl.ds(row, CPB)], w8)
        for b in range(CPB):
            pltpu.sync_copy(w8.at[b], dtab.at[d8.at[b]], add=True)
        return carry

    lax.fori_loop(0, CHUNKS_PER_TILE, chunk, 0)
    plsc.subcore_barrier()

    pltpu.sync_copy(dtab.at[pl.ds(s * SLICE, SLICE)],
                    deg_out.at[c].at[pl.ds(s * SLICE, SLICE)])


def _sc_degree(dstH, wH, z):
    return pl.kernel(
        _deg_body,
        out_type=jax.ShapeDtypeStruct((NC, NP), jnp.float32),
        mesh=_mesh,
        scratch_types=[
            pltpu.VMEM_SHARED((NP,), jnp.float32),
            pltpu.VMEM((CPB, BLK), jnp.int32),
            pltpu.VMEM((CPB, BLK), jnp.float32),
        ],
    )(dstH, wH, z)


def _agg_body(hsT, srcH, dstH, wH, z, S_out,
              h0, h1, h2, h3, o0, o1, o2, o3, s8, d8, w8, g, gm):
    c = lax.axis_index("c")
    s = lax.axis_index("s")
    wid = s * NC + c
    htab = [h0, h1, h2, h3]
    otab = [o0, o1, o2, o3]

    def pass_body(p, carry0):
        for f in range(F):
            pf = p * F + f
            pltpu.sync_copy(hsT.at[pf].at[pl.ds(s * SLICE, SLICE)],
                            htab[f].at[pl.ds(s * SLICE, SLICE)])
            pltpu.sync_copy(z, otab[f].at[pl.ds(s * SLICE, SLICE)])
        plsc.subcore_barrier()

        def chunk(k, carry):
            row = (wid * CHUNKS_PER_TILE + k) * CPB
            pltpu.sync_copy(srcH.at[pl.ds(row, CPB)], s8)
            pltpu.sync_copy(dstH.at[pl.ds(row, CPB)], d8)
            pltpu.sync_copy(wH.at[pl.ds(row, CPB)], w8)
            for b in range(CPB):
                for f in range(F):
                    pltpu.sync_copy(htab[f].at[s8.at[b]], g)
                    for l in range(BLK // 16):
                        sl = pl.ds(l * 16, 16)
                        gm[sl] = g[sl] * w8[b, sl]
                    pltpu.sync_copy(gm, otab[f].at[d8.at[b]], add=True)
            return carry

        lax.fori_loop(0, CHUNKS_PER_TILE, chunk, 0)
        plsc.subcore_barrier()

        for f in range(F):
            pf = p * F + f
            pltpu.sync_copy(otab[f].at[pl.ds(s * SLICE, SLICE)],
                            S_out.at[c * H + pf].at[pl.ds(s * SLICE, SLICE)])
        plsc.subcore_barrier()
        return carry0

    lax.fori_loop(0, P, pass_body, 0)


def _sc_edge_agg(hsT, srcH, dstH, wH, z):
    return pl.kernel(
        _agg_body,
        out_type=jax.ShapeDtypeStruct((NC * H, NP), jnp.float32),
        mesh=_mesh,
        scratch_types=[
            pltpu.VMEM_SHARED((NP,), jnp.float32),
            pltpu.VMEM_SHARED((NP,), jnp.float32),
            pltpu.VMEM_SHARED((NP,), jnp.float32),
            pltpu.VMEM_SHARED((NP,), jnp.float32),
            pltpu.VMEM_SHARED((NP,), jnp.float32),
            pltpu.VMEM_SHARED((NP,), jnp.float32),
            pltpu.VMEM_SHARED((NP,), jnp.float32),
            pltpu.VMEM_SHARED((NP,), jnp.float32),
            pltpu.VMEM((CPB, BLK), jnp.int32),
            pltpu.VMEM((CPB, BLK), jnp.int32),
            pltpu.VMEM((CPB, BLK), jnp.float32),
            pltpu.VMEM((BLK,), jnp.float32),
            pltpu.VMEM((BLK,), jnp.float32),
        ],
    )(hsT, srcH, dstH, wH, z)


_NB = 512


def _prescale_body(xT_ref, w_ref, deg_ref, hsT_ref):
    d = deg_ref[0, :] + deg_ref[1, :] + 1.0
    dinv = lax.rsqrt(d)
    h = jnp.dot(w_ref[...], xT_ref[...], preferred_element_type=jnp.float32)
    hsT_ref[...] = h * dinv[None, :]


def _tc_prescale(xT, W1T, deg):
    grid = (NP // _NB,)
    return pl.pallas_call(
        _prescale_body,
        grid=grid,
        in_specs=[
            pl.BlockSpec((IN, _NB), lambda i: (0, i)),
            pl.BlockSpec((H, IN), lambda i: (0, 0)),
            pl.BlockSpec((NC, _NB), lambda i: (0, i)),
        ],
        out_specs=pl.BlockSpec((H, _NB), lambda i: (0, i)),
        out_shape=jax.ShapeDtypeStruct((H, NP), jnp.float32),
    )(xT, W1T, deg)


def _layer_body(S_ref, hsT_ref, deg_ref, b_ref, w_ref, out_ref):
    d = deg_ref[0, :] + deg_ref[1, :] + 1.0
    dinv = lax.rsqrt(d)
    agg = S_ref[0, :, :] + S_ref[1, :, :] + hsT_ref[...]
    a = agg * dinv[None, :] + b_ref[...]
    h1 = a * jax.nn.sigmoid(a)
    out_ref[...] = jnp.dot(w_ref[...], h1,
                           preferred_element_type=jnp.float32) * dinv[None, :]


def _tc_layer(S, hsT, deg, b, W2T):
    grid = (NP // _NB,)
    return pl.pallas_call(
        _layer_body,
        grid=grid,
        in_specs=[
            pl.BlockSpec((NC, H, _NB), lambda i: (0, 0, i)),
            pl.BlockSpec((H, _NB), lambda i: (0, i)),
            pl.BlockSpec((NC, _NB), lambda i: (0, i)),
            pl.BlockSpec((H, 1), lambda i: (0, 0)),
            pl.BlockSpec((H, H), lambda i: (0, 0)),
        ],
        out_specs=pl.BlockSpec((H, _NB), lambda i: (0, i)),
        out_shape=jax.ShapeDtypeStruct((H, NP), jnp.float32),
    )(S, hsT, deg, b, W2T)


def _final_body(S_ref, hsT_ref, deg_ref, b2_ref, wh1_ref, bh1_ref,
                wh2_ref, bh2_ref, out_ref):
    d = deg_ref[0, :] + deg_ref[1, :] + 1.0
    dinv = lax.rsqrt(d)
    agg = S_ref[0, :, :] + S_ref[1, :, :] + hsT_ref[...]
    a = agg * dinv[None, :] + b2_ref[...]
    h2 = a * jax.nn.sigmoid(a)
    zz = jnp.dot(wh1_ref[...], h2,
                 preferred_element_type=jnp.float32) + bh1_ref[...]
    zs = zz * jax.nn.sigmoid(zz)
    out_ref[...] = jnp.dot(wh2_ref[...], zs,
                           preferred_element_type=jnp.float32) + bh2_ref[...]


def _tc_final(Sb, hsb, degb, b2, Wh1T, bh1, Wh2T, bh2):
    G = N // 3
    grid = (pl.cdiv(G, _NB),)
    return pl.pallas_call(
        _final_body,
        grid=grid,
        in_specs=[
            pl.BlockSpec((NC, H, _NB), lambda i: (0, 0, i)),
            pl.BlockSpec((H, _NB), lambda i: (0, i)),
            pl.BlockSpec((NC, _NB), lambda i: (0, i)),
            pl.BlockSpec((H, 1), lambda i: (0, 0)),
            pl.BlockSpec((H, H), lambda i: (0, 0)),
            pl.BlockSpec((H, 1), lambda i: (0, 0)),
            pl.BlockSpec((3, H), lambda i: (0, 0)),
            pl.BlockSpec((3, 1), lambda i: (0, 0)),
        ],
        out_specs=pl.BlockSpec((3, _NB), lambda i: (0, i)),
        out_shape=jax.ShapeDtypeStruct((3, G), jnp.float32),
    )(Sb, hsb, degb, b2, Wh1T, bh1, Wh2T, bh2)


@jax.jit
def kernel(x, edge_index, edge_weight, W1, b1, W2, b2, Wh1, bh1, Wh2, bh2):
    pad = E_PAD - E
    ar = jnp.arange(pad, dtype=jnp.int32)
    src_p = jnp.concatenate([edge_index[0], ar % jnp.int32(N)]).reshape(EROWS, BLK)
    dst_p = jnp.concatenate([edge_index[1], ar % jnp.int32(N)]).reshape(EROWS, BLK)
    w_p = jnp.concatenate(
        [edge_weight, jnp.zeros((pad,), jnp.float32)]).reshape(EROWS, BLK)

    xT = jnp.pad(x, ((0, NP - N), (0, 0))).T
    z = jnp.zeros((SLICE,), jnp.float32)

    deg = _sc_degree(dst_p, w_p, z)

    hsT1 = _tc_prescale(xT, W1.T, deg)
    S1 = _sc_edge_agg(hsT1, src_p, dst_p, w_p, z).reshape(NC, H, NP)
    hsT2 = _tc_layer(S1, hsT1, deg, b1.reshape(H, 1), W2.T)

    S2 = _sc_edge_agg(hsT2, src_p, dst_p, w_p, z).reshape(NC, H, NP)

    G = N // 3
    Sb = S2[:, :, :N].reshape(NC, H, G, 3)[:, :, :, 2]
    hsb = hsT2[:, :N].reshape(H, G, 3)[:, :, 2]
    degb = deg[:, :N].reshape(NC, G, 3)[:, :, 2]

    outT = _tc_final(Sb, hsb, degb, b2.reshape(H, 1),
                     Wh1.T, bh1.reshape(H, 1), Wh2.T, bh2.reshape(3, 1))
    return outT.T


# speedup vs baseline: 4.7372x; 1.0737x over previous
import jax
import jax.numpy as jnp
from jax import lax
from jax.experimental import pallas as pl
from jax.experimental.pallas import tpu as pltpu
from jax.experimental.pallas import tpu_sc as plsc

N = 99999
IN = 128
H = 48
E = 1599984

NC = 2
NS = 16
NW = NC * NS

BLK = 128
CPB = 8
CHUNK = CPB * BLK
CHUNKS_PER_TILE = 49
EDGES_PER_TILE = CHUNK * CHUNKS_PER_TILE
E_PAD = EDGES_PER_TILE * NW
EROWS = E_PAD // BLK

NP = 104448
SLICE = NP // NS

F = 8
P = H // F

_mesh = plsc.VectorSubcoreMesh(core_axis_name="c", subcore_axis_name="s")


def _deg_body(dstH, wH, z, deg_out, dtab, d8, w8):
    c = lax.axis_index("c")
    s = lax.axis_index("s")
    wid = s * NC + c
    pltpu.sync_copy(z, dtab.at[pl.ds(s * SLICE, SLICE)])
    plsc.subcore_barrier()

    def chunk(k, carry):
        row = (wid * CHUNKS_PER_TILE + k) * CPB
        pltpu.sync_copy(dstH.at[pl.ds(row, CPB)], d8)
        pltpu.sync_copy(wH.at[pl.ds(row, CPB)], w8)
        for b in range(CPB):
            pltpu.sync_copy(w8.at[b], dtab.at[d8.at[b]], add=True)
        return carry

    lax.fori_loop(0, CHUNKS_PER_TILE, chunk, 0)
    plsc.subcore_barrier()

    pltpu.sync_copy(dtab.at[pl.ds(s * SLICE, SLICE)],
                    deg_out.at[c].at[pl.ds(s * SLICE, SLICE)])


def _sc_degree(dstH, wH, z):
    return pl.kernel(
        _deg_body,
        out_type=jax.ShapeDtypeStruct((NC, NP), jnp.float32),
        mesh=_mesh,
        scratch_types=[
            pltpu.VMEM_SHARED((NP,), jnp.float32),
            pltpu.VMEM((CPB, BLK), jnp.int32),
            pltpu.VMEM((CPB, BLK), jnp.float32),
        ],
    )(dstH, wH, z)


def _agg_body(hsT, srcH, dstH, wH, z, S_out,
              h0, h1, h2, h3, h4, h5, h6, h7,
              o0, o1, o2, o3, o4, o5, o6, o7, s8, d8, w8, g, gm):
    c = lax.axis_index("c")
    s = lax.axis_index("s")
    wid = s * NC + c
    htab = [h0, h1, h2, h3, h4, h5, h6, h7]
    otab = [o0, o1, o2, o3, o4, o5, o6, o7]

    def pass_body(p, carry0):
        for f in range(F):
            pf = p * F + f
            pltpu.sync_copy(hsT.at[pf].at[pl.ds(s * SLICE, SLICE)],
                            htab[f].at[pl.ds(s * SLICE, SLICE)])
            pltpu.sync_copy(z, otab[f].at[pl.ds(s * SLICE, SLICE)])
        plsc.subcore_barrier()

        def chunk(k, carry):
            row = (wid * CHUNKS_PER_TILE + k) * CPB
            pltpu.sync_copy(srcH.at[pl.ds(row, CPB)], s8)
            pltpu.sync_copy(dstH.at[pl.ds(row, CPB)], d8)
            pltpu.sync_copy(wH.at[pl.ds(row, CPB)], w8)
            for b in range(CPB):
                for f in range(F):
                    pltpu.sync_copy(htab[f].at[s8.at[b]], g)
                    for l in range(BLK // 16):
                        sl = pl.ds(l * 16, 16)
                        gm[sl] = g[sl] * w8[b, sl]
                    pltpu.sync_copy(gm, otab[f].at[d8.at[b]], add=True)
            return carry

        lax.fori_loop(0, CHUNKS_PER_TILE, chunk, 0)
        plsc.subcore_barrier()

        for f in range(F):
            pf = p * F + f
            pltpu.sync_copy(otab[f].at[pl.ds(s * SLICE, SLICE)],
                            S_out.at[c * H + pf].at[pl.ds(s * SLICE, SLICE)])
        plsc.subcore_barrier()
        return carry0

    lax.fori_loop(0, P, pass_body, 0)


def _sc_edge_agg(hsT, srcH, dstH, wH, z):
    return pl.kernel(
        _agg_body,
        out_type=jax.ShapeDtypeStruct((NC * H, NP), jnp.float32),
        mesh=_mesh,
        scratch_types=(
            [pltpu.VMEM_SHARED((NP,), jnp.float32)] * (2 * F)
        ) + [
            pltpu.VMEM((CPB, BLK), jnp.int32),
            pltpu.VMEM((CPB, BLK), jnp.int32),
            pltpu.VMEM((CPB, BLK), jnp.float32),
            pltpu.VMEM((BLK,), jnp.float32),
            pltpu.VMEM((BLK,), jnp.float32),
        ],
    )(hsT, srcH, dstH, wH, z)


_NB = 512


def _prescale_body(xT_ref, w_ref, deg_ref, hsT_ref):
    d = deg_ref[0, :] + deg_ref[1, :] + 1.0
    dinv = lax.rsqrt(d)
    h = jnp.dot(w_ref[...], xT_ref[...], preferred_element_type=jnp.float32)
    hsT_ref[...] = h * dinv[None, :]


def _tc_prescale(xT, W1T, deg):
    grid = (NP // _NB,)
    return pl.pallas_call(
        _prescale_body,
        grid=grid,
        in_specs=[
            pl.BlockSpec((IN, _NB), lambda i: (0, i)),
            pl.BlockSpec((H, IN), lambda i: (0, 0)),
            pl.BlockSpec((NC, _NB), lambda i: (0, i)),
        ],
        out_specs=pl.BlockSpec((H, _NB), lambda i: (0, i)),
        out_shape=jax.ShapeDtypeStruct((H, NP), jnp.float32),
    )(xT, W1T, deg)


def _layer_body(S_ref, hsT_ref, deg_ref, b_ref, w_ref, out_ref):
    d = deg_ref[0, :] + deg_ref[1, :] + 1.0
    dinv = lax.rsqrt(d)
    agg = S_ref[0, :, :] + S_ref[1, :, :] + hsT_ref[...]
    a = agg * dinv[None, :] + b_ref[...]
    h1 = a * jax.nn.sigmoid(a)
    out_ref[...] = jnp.dot(w_ref[...], h1,
                           preferred_element_type=jnp.float32) * dinv[None, :]


def _tc_layer(S, hsT, deg, b, W2T):
    grid = (NP // _NB,)
    return pl.pallas_call(
        _layer_body,
        grid=grid,
        in_specs=[
            pl.BlockSpec((NC, H, _NB), lambda i: (0, 0, i)),
            pl.BlockSpec((H, _NB), lambda i: (0, i)),
            pl.BlockSpec((NC, _NB), lambda i: (0, i)),
            pl.BlockSpec((H, 1), lambda i: (0, 0)),
            pl.BlockSpec((H, H), lambda i: (0, 0)),
        ],
        out_specs=pl.BlockSpec((H, _NB), lambda i: (0, i)),
        out_shape=jax.ShapeDtypeStruct((H, NP), jnp.float32),
    )(S, hsT, deg, b, W2T)


def _final_body(S_ref, hsT_ref, deg_ref, b2_ref, wh1_ref, bh1_ref,
                wh2_ref, bh2_ref, out_ref):
    d = deg_ref[0, :] + deg_ref[1, :] + 1.0
    dinv = lax.rsqrt(d)
    agg = S_ref[0, :, :] + S_ref[1, :, :] + hsT_ref[...]
    a = agg * dinv[None, :] + b2_ref[...]
    h2 = a * jax.nn.sigmoid(a)
    zz = jnp.dot(wh1_ref[...], h2,
                 preferred_element_type=jnp.float32) + bh1_ref[...]
    zs = zz * jax.nn.sigmoid(zz)
    out_ref[...] = jnp.dot(wh2_ref[...], zs,
                           preferred_element_type=jnp.float32) + bh2_ref[...]


def _tc_final(Sb, hsb, degb, b2, Wh1T, bh1, Wh2T, bh2):
    G = N // 3
    grid = (pl.cdiv(G, _NB),)
    return pl.pallas_call(
        _final_body,
        grid=grid,
        in_specs=[
            pl.BlockSpec((NC, H, _NB), lambda i: (0, 0, i)),
            pl.BlockSpec((H, _NB), lambda i: (0, i)),
            pl.BlockSpec((NC, _NB), lambda i: (0, i)),
            pl.BlockSpec((H, 1), lambda i: (0, 0)),
            pl.BlockSpec((H, H), lambda i: (0, 0)),
            pl.BlockSpec((H, 1), lambda i: (0, 0)),
            pl.BlockSpec((3, H), lambda i: (0, 0)),
            pl.BlockSpec((3, 1), lambda i: (0, 0)),
        ],
        out_specs=pl.BlockSpec((3, _NB), lambda i: (0, i)),
        out_shape=jax.ShapeDtypeStruct((3, G), jnp.float32),
    )(Sb, hsb, degb, b2, Wh1T, bh1, Wh2T, bh2)


@jax.jit
def kernel(x, edge_index, edge_weight, W1, b1, W2, b2, Wh1, bh1, Wh2, bh2):
    pad = E_PAD - E
    ar = jnp.arange(pad, dtype=jnp.int32)
    src_p = jnp.concatenate([edge_index[0], ar % jnp.int32(N)]).reshape(EROWS, BLK)
    dst_p = jnp.concatenate([edge_index[1], ar % jnp.int32(N)]).reshape(EROWS, BLK)
    w_p = jnp.concatenate(
        [edge_weight, jnp.zeros((pad,), jnp.float32)]).reshape(EROWS, BLK)

    xT = jnp.pad(x, ((0, NP - N), (0, 0))).T
    z = jnp.zeros((SLICE,), jnp.float32)

    deg = _sc_degree(dst_p, w_p, z)

    hsT1 = _tc_prescale(xT, W1.T, deg)
    S1 = _sc_edge_agg(hsT1, src_p, dst_p, w_p, z).reshape(NC, H, NP)
    hsT2 = _tc_layer(S1, hsT1, deg, b1.reshape(H, 1), W2.T)

    S2 = _sc_edge_agg(hsT2, src_p, dst_p, w_p, z).reshape(NC, H, NP)

    G = N // 3
    Sb = S2[:, :, :N].reshape(NC, H, G, 3)[:, :, :, 2]
    hsb = hsT2[:, :N].reshape(H, G, 3)[:, :, 2]
    degb = deg[:, :N].reshape(NC, G, 3)[:, :, 2]

    outT = _tc_final(Sb, hsb, degb, b2.reshape(H, 1),
                     Wh1.T, bh1.reshape(H, 1), Wh2.T, bh2.reshape(3, 1))
    return outT.T
